# baseline (speedup 1.0000x reference)
import jax
import jax.numpy as jnp
from jax.experimental import pallas as pl
from jax.experimental.pallas import tpu as pltpu

_MU = 3.5
_N_ITEMS = 2048
_N_USERS = 2048
_BLK = 256
_NBLK = _N_USERS // _BLK


def _nbm_kernel(sref, r_blk, r_urow, w_row, o_row, ib_row, ub_row, out,
                acc_ss, safe_ref, acc_dot):
    g = pl.program_id(0)
    u = sref[0]
    i = sref[1]
    kval = sref[2]
    lane = jax.lax.broadcasted_iota(jnp.int32, (1, _N_ITEMS), 1)

    @pl.when(g == 0)
    def _init():
        acc_ss[...] = jnp.zeros_like(acc_ss)
        acc_dot[...] = jnp.zeros_like(acc_dot)

    blk = r_blk[...]

    @pl.when(g < _NBLK)
    def _pass_a():
        acc_ss[...] += jnp.sum(blk * blk, axis=0, keepdims=True)

    @pl.when(g == _NBLK)
    def _mid():
        norm = jnp.sqrt(acc_ss[...])
        safe_ref[...] = jnp.where(norm == 0.0, 1.0, norm)

    @pl.when(g >= _NBLK)
    def _pass_b():
        safe = safe_ref[...]
        blkn = blk / safe
        oh = (lane == i).astype(jnp.float32)
        cn = jax.lax.dot_general(blkn, oh, (((1,), (1,)), ((), ())),
                                 precision=jax.lax.Precision.HIGHEST,
                                 preferred_element_type=jnp.float32)
        acc_dot[...] += jax.lax.dot_general(cn, blkn,
                                            (((0,), (0,)), ((), ())),
                                            preferred_element_type=jnp.float32)

    @pl.when(g == 2 * _NBLK - 1)
    def _finalize():
        s_row = acc_dot[...]

        r2, c2 = 16, _N_ITEMS // 16
        s2d = jnp.reshape(s_row, (r2, c2))
        fidx = (jax.lax.broadcasted_iota(jnp.int32, (r2, c2), 0) * c2
                + jax.lax.broadcasted_iota(jnp.int32, (r2, c2), 1))

        bits = jax.lax.bitcast_convert_type(s2d, jnp.int32)
        skey = bits ^ (jnp.right_shift(bits, 31) & jnp.int32(0x7FFFFFFF))
        thr = jnp.int32(-(1 << 31))
        for b in range(31, -1, -1):
            step = (1 << b) - (1 << 32) if b == 31 else (1 << b)
            cand = thr + jnp.int32(step)
            cnt = jnp.sum((skey >= cand).astype(jnp.int32))
            thr = jnp.where(cnt >= kval, cand, thr)
        gt = skey > thr
        needed = kval - jnp.sum(gt.astype(jnp.int32))
        eq = skey == thr
        jcut = jnp.int32(0)
        for b in range(11, -1, -1):
            candj = jcut + jnp.int32(1 << b)
            cntj = jnp.sum((eq & (fidx < candj)).astype(jnp.int32))
            jcut = jnp.where(cntj <= needed, candj, jcut)
        topk = gt | (eq & (fidx < jcut))

        ru = jnp.reshape(r_urow[0], (r2, c2))
        w2 = jnp.reshape(w_row[0], (r2, c2))
        o2 = jnp.reshape(o_row[0], (r2, c2))
        ib2 = jnp.reshape(ib_row[...], (r2, c2))
        ub2 = jnp.reshape(ub_row[...], (r2, c2))
        selm = topk & (ru != 0.0)
        num_k = jnp.sum(selm.astype(jnp.float32))
        bu = jnp.sum(jnp.where(fidx == u, ub2, 0.0))
        bi = jnp.sum(jnp.where(fidx == i, ib2, 0.0))
        buj = _MU + bu + ib2
        ruj = jnp.floor(ru)
        s1 = jnp.sum(jnp.where(selm, (ruj - buj) * w2, 0.0))
        s2 = jnp.sum(jnp.where(selm, o2, 0.0))
        nrm = jax.lax.rsqrt(num_k)
        rui = _MU + bu + bi + nrm * s1 + nrm * s2
        out[...] = jnp.reshape(rui, (1, 1))


def kernel(R, user, item, item_weights, implicit_offset, user_biases,
           item_biases, k):
    u32 = user.astype(jnp.int32)[0]
    i32 = item.astype(jnp.int32)[0]
    k32 = jnp.asarray(k, jnp.int32)
    sref = jnp.stack([u32, i32, k32])
    ib = item_biases.reshape(1, _N_ITEMS)
    ub = user_biases.reshape(1, _N_USERS)
    R3 = R.reshape(_N_USERS, 1, _N_ITEMS)
    W3 = item_weights.reshape(_N_ITEMS, 1, _N_ITEMS)
    O3 = implicit_offset.reshape(_N_ITEMS, 1, _N_ITEMS)

    grid_spec = pltpu.PrefetchScalarGridSpec(
        num_scalar_prefetch=1,
        grid=(2 * _NBLK,),
        in_specs=[
            pl.BlockSpec((_BLK, _N_ITEMS), lambda g, s: (g % _NBLK, 0)),
            pl.BlockSpec((1, 1, _N_ITEMS), lambda g, s: (s[0], 0, 0)),
            pl.BlockSpec((1, 1, _N_ITEMS), lambda g, s: (s[1], 0, 0)),
            pl.BlockSpec((1, 1, _N_ITEMS), lambda g, s: (s[1], 0, 0)),
            pl.BlockSpec((1, _N_ITEMS), lambda g, s: (0, 0)),
            pl.BlockSpec((1, _N_USERS), lambda g, s: (0, 0)),
        ],
        out_specs=pl.BlockSpec((1, 1), lambda g, s: (0, 0)),
        scratch_shapes=[pltpu.VMEM((1, _N_ITEMS), jnp.float32)] * 3,
    )
    out = pl.pallas_call(
        _nbm_kernel,
        grid_spec=grid_spec,
        out_shape=jax.ShapeDtypeStruct((1, 1), jnp.float32),
    )(sref, R, R3, W3, O3, ib, ub)
    return out[0, 0]


# speedup vs baseline: 2.5244x; 2.5244x over previous
import jax
import jax.numpy as jnp
from jax.experimental import pallas as pl
from jax.experimental.pallas import tpu as pltpu

_MU = 3.5
_N_ITEMS = 2048
_N_USERS = 2048
_BLK = 256
_NBLK = _N_USERS // _BLK


def _nbm_kernel(sref, r_blk, r_urow, w_row, o_row, ib_row, ub_row, out,
                acc_ss, safe_ref, acc_dot):
    g = pl.program_id(0)
    u = sref[0]
    i = sref[1]
    kval = sref[2]
    lane = jax.lax.broadcasted_iota(jnp.int32, (1, _N_ITEMS), 1)

    @pl.when(g == 0)
    def _init():
        acc_ss[...] = jnp.zeros_like(acc_ss)
        acc_dot[...] = jnp.zeros_like(acc_dot)

    blk = r_blk[...]

    @pl.when(g < _NBLK)
    def _pass_a():
        acc_ss[...] += jnp.sum(blk * blk, axis=0, keepdims=True)

    @pl.when(g == _NBLK)
    def _mid():
        norm = jnp.sqrt(acc_ss[...])
        safe_ref[...] = jnp.where(norm == 0.0, 1.0, norm)

    @pl.when(g >= _NBLK)
    def _pass_b():
        safe = safe_ref[...]
        blkn = blk / safe
        oh = (lane == i).astype(jnp.float32)
        cn = jax.lax.dot_general(blkn, oh, (((1,), (1,)), ((), ())),
                                 precision=jax.lax.Precision.HIGHEST,
                                 preferred_element_type=jnp.float32)
        acc_dot[...] += jax.lax.dot_general(cn, blkn,
                                            (((0,), (0,)), ((), ())),
                                            preferred_element_type=jnp.float32)

    @pl.when(g == 2 * _NBLK - 1)
    def _finalize():
        s_row = acc_dot[...]

        r2, c2 = 16, _N_ITEMS // 16
        s2d = jnp.reshape(s_row, (r2, c2))
        fidx = (jax.lax.broadcasted_iota(jnp.int32, (r2, c2), 0) * c2
                + jax.lax.broadcasted_iota(jnp.int32, (r2, c2), 1))

        bits = jax.lax.bitcast_convert_type(s2d, jnp.int32)
        skey = bits ^ (jnp.right_shift(bits, 31) & jnp.int32(0x7FFFFFFF))
        thr = jnp.int32(-(1 << 31))
        for b in range(31, -1, -1):
            step = (1 << b) - (1 << 32) if b == 31 else (1 << b)
            cand = thr + jnp.int32(step)
            cnt = jnp.sum((skey >= cand).astype(jnp.int32))
            thr = jnp.where(cnt >= kval, cand, thr)
        gt = skey > thr
        needed = kval - jnp.sum(gt.astype(jnp.int32))
        eq = skey == thr
        jcut = jnp.int32(0)
        for b in range(11, -1, -1):
            candj = jcut + jnp.int32(1 << b)
            cntj = jnp.sum((eq & (fidx < candj)).astype(jnp.int32))
            jcut = jnp.where(cntj <= needed, candj, jcut)
        topk = gt | (eq & (fidx < jcut))

        sub = jax.lax.broadcasted_iota(jnp.int32, (8, _N_ITEMS), 0)
        ru1 = jnp.sum(jnp.where(sub == (u & 7), r_urow[...], 0.0),
                      axis=0, keepdims=True)
        w1 = jnp.sum(jnp.where(sub == (i & 7), w_row[...], 0.0),
                     axis=0, keepdims=True)
        o1 = jnp.sum(jnp.where(sub == (i & 7), o_row[...], 0.0),
                     axis=0, keepdims=True)
        ru = jnp.reshape(ru1, (r2, c2))
        w2 = jnp.reshape(w1, (r2, c2))
        o2 = jnp.reshape(o1, (r2, c2))
        ib2 = jnp.reshape(ib_row[...], (r2, c2))
        ub2 = jnp.reshape(ub_row[...], (r2, c2))
        selm = topk & (ru != 0.0)
        num_k = jnp.sum(selm.astype(jnp.float32))
        bu = jnp.sum(jnp.where(fidx == u, ub2, 0.0))
        bi = jnp.sum(jnp.where(fidx == i, ib2, 0.0))
        buj = _MU + bu + ib2
        ruj = jnp.floor(ru)
        s1 = jnp.sum(jnp.where(selm, (ruj - buj) * w2, 0.0))
        s2 = jnp.sum(jnp.where(selm, o2, 0.0))
        nrm = jax.lax.rsqrt(num_k)
        rui = _MU + bu + bi + nrm * s1 + nrm * s2
        out[...] = jnp.reshape(rui, (1, 1))


def kernel(R, user, item, item_weights, implicit_offset, user_biases,
           item_biases, k):
    u32 = user.astype(jnp.int32)[0]
    i32 = item.astype(jnp.int32)[0]
    k32 = jnp.asarray(k, jnp.int32)
    sref = jnp.stack([u32, i32, k32])
    ib = item_biases.reshape(1, _N_ITEMS)
    ub = user_biases.reshape(1, _N_USERS)

    grid_spec = pltpu.PrefetchScalarGridSpec(
        num_scalar_prefetch=1,
        grid=(2 * _NBLK,),
        in_specs=[
            pl.BlockSpec((_BLK, _N_ITEMS), lambda g, s: (g % _NBLK, 0)),
            pl.BlockSpec((8, _N_ITEMS), lambda g, s: (s[0] // 8, 0)),
            pl.BlockSpec((8, _N_ITEMS), lambda g, s: (s[1] // 8, 0)),
            pl.BlockSpec((8, _N_ITEMS), lambda g, s: (s[1] // 8, 0)),
            pl.BlockSpec((1, _N_ITEMS), lambda g, s: (0, 0)),
            pl.BlockSpec((1, _N_USERS), lambda g, s: (0, 0)),
        ],
        out_specs=pl.BlockSpec((1, 1), lambda g, s: (0, 0)),
        scratch_shapes=[pltpu.VMEM((1, _N_ITEMS), jnp.float32)] * 3,
    )
    out = pl.pallas_call(
        _nbm_kernel,
        grid_spec=grid_spec,
        out_shape=jax.ShapeDtypeStruct((1, 1), jnp.float32),
    )(sref, R, R, item_weights, implicit_offset, ib, ub)
    return out[0, 0]


# speedup vs baseline: 3.2824x; 1.3003x over previous
import jax
import jax.numpy as jnp
from jax.experimental import pallas as pl
from jax.experimental.pallas import tpu as pltpu

_MU = 3.5
_N_ITEMS = 2048
_N_USERS = 2048
_CB = 256
_NBLK = _N_ITEMS // _CB


def _nbm_kernel(sref, r_blk, r_urow, w_row, o_row, ib_row, ub_row, out,
                rn_scr):
    g = pl.program_id(0)
    u = sref[0]
    i = sref[1]
    kval = sref[2]

    blk = r_blk[...]
    ss = jnp.sum(blk * blk, axis=0, keepdims=True)
    norm = jnp.sqrt(ss)
    safe = jnp.where(norm == 0.0, 1.0, norm)
    rn_scr[:, pl.ds(pl.multiple_of(g * _CB, _CB), _CB)] = blk / safe

    @pl.when(g == _NBLK - 1)
    def _finalize():
        lane = jax.lax.broadcasted_iota(jnp.int32, (1, _N_ITEMS), 1)
        oh = (lane == i).astype(jnp.float32)
        acc = jnp.zeros((1, _N_ITEMS), jnp.float32)
        for t in range(_N_USERS // _CB):
            slab = rn_scr[pl.ds(t * _CB, _CB), :]
            cn = jax.lax.dot_general(slab, oh, (((1,), (1,)), ((), ())),
                                     precision=jax.lax.Precision.HIGHEST,
                                     preferred_element_type=jnp.float32)
            acc = acc + jax.lax.dot_general(cn, slab,
                                            (((0,), (0,)), ((), ())),
                                            preferred_element_type=jnp.float32)
        s_row = acc

        r2, c2 = 16, _N_ITEMS // 16
        s2d = jnp.reshape(s_row, (r2, c2))
        fidx = (jax.lax.broadcasted_iota(jnp.int32, (r2, c2), 0) * c2
                + jax.lax.broadcasted_iota(jnp.int32, (r2, c2), 1))

        bits = jax.lax.bitcast_convert_type(s2d, jnp.int32)
        skey = bits ^ (jnp.right_shift(bits, 31) & jnp.int32(0x7FFFFFFF))
        thr = jnp.int32(-(1 << 31))
        for b in range(31, -1, -1):
            step = (1 << b) - (1 << 32) if b == 31 else (1 << b)
            cand = thr + jnp.int32(step)
            cnt = jnp.sum((skey >= cand).astype(jnp.int32))
            thr = jnp.where(cnt >= kval, cand, thr)
        gt = skey > thr
        needed = kval - jnp.sum(gt.astype(jnp.int32))
        eq = skey == thr
        jcut = jnp.int32(0)
        for b in range(11, -1, -1):
            candj = jcut + jnp.int32(1 << b)
            cntj = jnp.sum((eq & (fidx < candj)).astype(jnp.int32))
            jcut = jnp.where(cntj <= needed, candj, jcut)
        topk = gt | (eq & (fidx < jcut))

        sub = jax.lax.broadcasted_iota(jnp.int32, (8, _N_ITEMS), 0)
        ru1 = jnp.sum(jnp.where(sub == (u & 7), r_urow[...], 0.0),
                      axis=0, keepdims=True)
        w1 = jnp.sum(jnp.where(sub == (i & 7), w_row[...], 0.0),
                     axis=0, keepdims=True)
        o1 = jnp.sum(jnp.where(sub == (i & 7), o_row[...], 0.0),
                     axis=0, keepdims=True)
        ru = jnp.reshape(ru1, (r2, c2))
        w2 = jnp.reshape(w1, (r2, c2))
        o2 = jnp.reshape(o1, (r2, c2))
        ib2 = jnp.reshape(ib_row[...], (r2, c2))
        ub2 = jnp.reshape(ub_row[...], (r2, c2))
        selm = topk & (ru != 0.0)
        num_k = jnp.sum(selm.astype(jnp.float32))
        bu = jnp.sum(jnp.where(fidx == u, ub2, 0.0))
        bi = jnp.sum(jnp.where(fidx == i, ib2, 0.0))
        buj = _MU + bu + ib2
        ruj = jnp.floor(ru)
        s1 = jnp.sum(jnp.where(selm, (ruj - buj) * w2, 0.0))
        s2 = jnp.sum(jnp.where(selm, o2, 0.0))
        nrm = jax.lax.rsqrt(num_k)
        rui = _MU + bu + bi + nrm * s1 + nrm * s2
        out[...] = jnp.reshape(rui, (1, 1))


def kernel(R, user, item, item_weights, implicit_offset, user_biases,
           item_biases, k):
    u32 = user.astype(jnp.int32)[0]
    i32 = item.astype(jnp.int32)[0]
    k32 = jnp.asarray(k, jnp.int32)
    sref = jnp.stack([u32, i32, k32])
    ib = item_biases.reshape(1, _N_ITEMS)
    ub = user_biases.reshape(1, _N_USERS)

    grid_spec = pltpu.PrefetchScalarGridSpec(
        num_scalar_prefetch=1,
        grid=(_NBLK,),
        in_specs=[
            pl.BlockSpec((_N_USERS, _CB), lambda g, s: (0, g)),
            pl.BlockSpec((8, _N_ITEMS), lambda g, s: (s[0] // 8, 0)),
            pl.BlockSpec((8, _N_ITEMS), lambda g, s: (s[1] // 8, 0)),
            pl.BlockSpec((8, _N_ITEMS), lambda g, s: (s[1] // 8, 0)),
            pl.BlockSpec((1, _N_ITEMS), lambda g, s: (0, 0)),
            pl.BlockSpec((1, _N_USERS), lambda g, s: (0, 0)),
        ],
        out_specs=pl.BlockSpec((1, 1), lambda g, s: (0, 0)),
        scratch_shapes=[pltpu.VMEM((_N_USERS, _N_ITEMS), jnp.float32)],
    )
    out = pl.pallas_call(
        _nbm_kernel,
        grid_spec=grid_spec,
        out_shape=jax.ShapeDtypeStruct((1, 1), jnp.float32),
    )(sref, R, R, item_weights, implicit_offset, ib, ub)
    return out[0, 0]


# speedup vs baseline: 3.3736x; 1.0278x over previous
import jax
import jax.numpy as jnp
from jax.experimental import pallas as pl
from jax.experimental.pallas import tpu as pltpu

_MU = 3.5
_N_ITEMS = 2048
_N_USERS = 2048
_CB = 256
_NBLK = _N_ITEMS // _CB


def _nbm_kernel(sref, r_blk, r_urow, w_row, o_row, ib_row, ub_row, out,
                rn_scr, cn_scr):
    g = pl.program_id(0)
    u = sref[0]
    i = sref[1]
    kval = sref[2]

    blk = r_blk[...]
    ss = jnp.sum(blk * blk, axis=0, keepdims=True)
    norm = jnp.sqrt(ss)
    safe = jnp.where(norm == 0.0, 1.0, norm)
    rn_scr[:, pl.ds(pl.multiple_of(g * _CB, _CB), _CB)] = blk / safe

    @pl.when(g == i // _CB)
    def _extract():
        lane = jax.lax.broadcasted_iota(jnp.int32, (1, _CB), 1)
        ohl = (lane == i % _CB).astype(jnp.float32)
        si = jnp.sum(jnp.where(lane == i % _CB, safe, 0.0))
        for t in range(_N_USERS // _CB):
            bt = jax.lax.slice(blk, (t * _CB, 0), ((t + 1) * _CB, _CB))
            c_t = jax.lax.dot_general(bt, ohl, (((1,), (1,)), ((), ())),
                                      preferred_element_type=jnp.float32)
            cn_scr[pl.ds(t * _CB, _CB), :] = c_t / si

    @pl.when(g == _NBLK - 1)
    def _finalize():
        acc = jnp.zeros((1, _N_ITEMS), jnp.float32)
        for t in range(_N_USERS // _CB):
            slab = rn_scr[pl.ds(t * _CB, _CB), :]
            cn = cn_scr[pl.ds(t * _CB, _CB), :]
            acc = acc + jax.lax.dot_general(cn, slab,
                                            (((0,), (0,)), ((), ())),
                                            preferred_element_type=jnp.float32)
        s_row = acc

        r2, c2 = 16, _N_ITEMS // 16
        s2d = jnp.reshape(s_row, (r2, c2))
        fidx = (jax.lax.broadcasted_iota(jnp.int32, (r2, c2), 0) * c2
                + jax.lax.broadcasted_iota(jnp.int32, (r2, c2), 1))

        bits = jax.lax.bitcast_convert_type(s2d, jnp.int32)
        skey = bits ^ (jnp.right_shift(bits, 31) & jnp.int32(0x7FFFFFFF))
        thr = jnp.int32(-(1 << 31))
        for b in range(31, -1, -1):
            step = (1 << b) - (1 << 32) if b == 31 else (1 << b)
            cand = thr + jnp.int32(step)
            cnt = jnp.sum((skey >= cand).astype(jnp.int32))
            thr = jnp.where(cnt >= kval, cand, thr)
        gt = skey > thr
        needed = kval - jnp.sum(gt.astype(jnp.int32))
        eq = skey == thr
        jcut = jnp.int32(0)
        for b in range(11, -1, -1):
            candj = jcut + jnp.int32(1 << b)
            cntj = jnp.sum((eq & (fidx < candj)).astype(jnp.int32))
            jcut = jnp.where(cntj <= needed, candj, jcut)
        topk = gt | (eq & (fidx < jcut))

        sub = jax.lax.broadcasted_iota(jnp.int32, (8, _N_ITEMS), 0)
        ru1 = jnp.sum(jnp.where(sub == (u & 7), r_urow[...], 0.0),
                      axis=0, keepdims=True)
        w1 = jnp.sum(jnp.where(sub == (i & 7), w_row[...], 0.0),
                     axis=0, keepdims=True)
        o1 = jnp.sum(jnp.where(sub == (i & 7), o_row[...], 0.0),
                     axis=0, keepdims=True)
        ru = jnp.reshape(ru1, (r2, c2))
        w2 = jnp.reshape(w1, (r2, c2))
        o2 = jnp.reshape(o1, (r2, c2))
        ib2 = jnp.reshape(ib_row[...], (r2, c2))
        ub2 = jnp.reshape(ub_row[...], (r2, c2))
        selm = topk & (ru != 0.0)
        num_k = jnp.sum(selm.astype(jnp.float32))
        bu = jnp.sum(jnp.where(fidx == u, ub2, 0.0))
        bi = jnp.sum(jnp.where(fidx == i, ib2, 0.0))
        buj = _MU + bu + ib2
        ruj = jnp.floor(ru)
        s1 = jnp.sum(jnp.where(selm, (ruj - buj) * w2, 0.0))
        s2 = jnp.sum(jnp.where(selm, o2, 0.0))
        nrm = jax.lax.rsqrt(num_k)
        rui = _MU + bu + bi + nrm * s1 + nrm * s2
        out[...] = jnp.reshape(rui, (1, 1))


def kernel(R, user, item, item_weights, implicit_offset, user_biases,
           item_biases, k):
    u32 = user.astype(jnp.int32)[0]
    i32 = item.astype(jnp.int32)[0]
    k32 = jnp.asarray(k, jnp.int32)
    sref = jnp.stack([u32, i32, k32])
    ib = item_biases.reshape(1, _N_ITEMS)
    ub = user_biases.reshape(1, _N_USERS)

    grid_spec = pltpu.PrefetchScalarGridSpec(
        num_scalar_prefetch=1,
        grid=(_NBLK,),
        in_specs=[
            pl.BlockSpec((_N_USERS, _CB), lambda g, s: (0, g)),
            pl.BlockSpec((8, _N_ITEMS), lambda g, s: (s[0] // 8, 0)),
            pl.BlockSpec((8, _N_ITEMS), lambda g, s: (s[1] // 8, 0)),
            pl.BlockSpec((8, _N_ITEMS), lambda g, s: (s[1] // 8, 0)),
            pl.BlockSpec((1, _N_ITEMS), lambda g, s: (0, 0)),
            pl.BlockSpec((1, _N_USERS), lambda g, s: (0, 0)),
        ],
        out_specs=pl.BlockSpec((1, 1), lambda g, s: (0, 0)),
        scratch_shapes=[pltpu.VMEM((_N_USERS, _N_ITEMS), jnp.float32),
                        pltpu.VMEM((_N_USERS, 1), jnp.float32)],
    )
    out = pl.pallas_call(
        _nbm_kernel,
        grid_spec=grid_spec,
        out_shape=jax.ShapeDtypeStruct((1, 1), jnp.float32),
    )(sref, R, R, item_weights, implicit_offset, ib, ub)
    return out[0, 0]


# speedup vs baseline: 3.5006x; 1.0376x over previous
import jax
import jax.numpy as jnp
from jax.experimental import pallas as pl
from jax.experimental.pallas import tpu as pltpu

_MU = 3.5
_N_ITEMS = 2048
_N_USERS = 2048
_CB = 256
_NBLK = _N_ITEMS // _CB


def _nbm_kernel(sref, r_blk, r_urow, w_row, o_row, ib_row, ub_row, out,
                rn_scr, cn_scr):
    g = pl.program_id(0)
    u = sref[0]
    i = sref[1]
    kval = sref[2]

    blk = r_blk[...]
    ss = jnp.sum(blk * blk, axis=0, keepdims=True)
    norm = jnp.sqrt(ss)
    safe = jnp.where(norm == 0.0, 1.0, norm)
    rn_scr[:, pl.ds(pl.multiple_of(g * _CB, _CB), _CB)] = (
        (blk / safe).astype(jnp.bfloat16))

    @pl.when(g == i // _CB)
    def _extract():
        lane = jax.lax.broadcasted_iota(jnp.int32, (1, _CB), 1)
        ohl = (lane == i % _CB).astype(jnp.float32)
        si = jnp.sum(jnp.where(lane == i % _CB, safe, 0.0))
        for t in range(_N_USERS // _CB):
            bt = jax.lax.slice(blk, (t * _CB, 0), ((t + 1) * _CB, _CB))
            c_t = jax.lax.dot_general(bt, ohl, (((1,), (1,)), ((), ())),
                                      preferred_element_type=jnp.float32)
            cn_scr[pl.ds(t * _CB, _CB), :] = c_t / si

    @pl.when(g == _NBLK - 1)
    def _finalize():
        acc = jnp.zeros((1, _N_ITEMS), jnp.float32)
        for t in range(_N_USERS // _CB):
            slab = rn_scr[pl.ds(t * _CB, _CB), :]
            cn = cn_scr[pl.ds(t * _CB, _CB), :].astype(jnp.bfloat16)
            acc = acc + jax.lax.dot_general(cn, slab,
                                            (((0,), (0,)), ((), ())),
                                            preferred_element_type=jnp.float32)
        s_row = acc

        r2, c2 = 16, _N_ITEMS // 16
        s2d = jnp.reshape(s_row, (r2, c2))
        fidx = (jax.lax.broadcasted_iota(jnp.int32, (r2, c2), 0) * c2
                + jax.lax.broadcasted_iota(jnp.int32, (r2, c2), 1))

        bits = jax.lax.bitcast_convert_type(s2d, jnp.int32)
        skey = bits ^ (jnp.right_shift(bits, 31) & jnp.int32(0x7FFFFFFF))
        thr = jnp.int32(-(1 << 31))
        for b in range(31, -1, -1):
            step = (1 << b) - (1 << 32) if b == 31 else (1 << b)
            cand = thr + jnp.int32(step)
            cnt = jnp.sum((skey >= cand).astype(jnp.int32))
            thr = jnp.where(cnt >= kval, cand, thr)
        gt = skey > thr
        needed = kval - jnp.sum(gt.astype(jnp.int32))
        eq = skey == thr
        jcut = jnp.int32(0)
        for b in range(11, -1, -1):
            candj = jcut + jnp.int32(1 << b)
            cntj = jnp.sum((eq & (fidx < candj)).astype(jnp.int32))
            jcut = jnp.where(cntj <= needed, candj, jcut)
        topk = gt | (eq & (fidx < jcut))

        sub = jax.lax.broadcasted_iota(jnp.int32, (8, _N_ITEMS), 0)
        ru1 = jnp.sum(jnp.where(sub == (u & 7), r_urow[...], 0.0),
                      axis=0, keepdims=True)
        w1 = jnp.sum(jnp.where(sub == (i & 7), w_row[...], 0.0),
                     axis=0, keepdims=True)
        o1 = jnp.sum(jnp.where(sub == (i & 7), o_row[...], 0.0),
                     axis=0, keepdims=True)
        ru = jnp.reshape(ru1, (r2, c2))
        w2 = jnp.reshape(w1, (r2, c2))
        o2 = jnp.reshape(o1, (r2, c2))
        ib2 = jnp.reshape(ib_row[...], (r2, c2))
        ub2 = jnp.reshape(ub_row[...], (r2, c2))
        selm = topk & (ru != 0.0)
        num_k = jnp.sum(selm.astype(jnp.float32))
        bu = jnp.sum(jnp.where(fidx == u, ub2, 0.0))
        bi = jnp.sum(jnp.where(fidx == i, ib2, 0.0))
        buj = _MU + bu + ib2
        ruj = jnp.floor(ru)
        s1 = jnp.sum(jnp.where(selm, (ruj - buj) * w2, 0.0))
        s2 = jnp.sum(jnp.where(selm, o2, 0.0))
        nrm = jax.lax.rsqrt(num_k)
        rui = _MU + bu + bi + nrm * s1 + nrm * s2
        out[...] = jnp.reshape(rui, (1, 1))


def kernel(R, user, item, item_weights, implicit_offset, user_biases,
           item_biases, k):
    u32 = user.astype(jnp.int32)[0]
    i32 = item.astype(jnp.int32)[0]
    k32 = jnp.asarray(k, jnp.int32)
    sref = jnp.stack([u32, i32, k32])
    ib = item_biases.reshape(1, _N_ITEMS)
    ub = user_biases.reshape(1, _N_USERS)

    grid_spec = pltpu.PrefetchScalarGridSpec(
        num_scalar_prefetch=1,
        grid=(_NBLK,),
        in_specs=[
            pl.BlockSpec((_N_USERS, _CB), lambda g, s: (0, g)),
            pl.BlockSpec((8, _N_ITEMS), lambda g, s: (s[0] // 8, 0)),
            pl.BlockSpec((8, _N_ITEMS), lambda g, s: (s[1] // 8, 0)),
            pl.BlockSpec((8, _N_ITEMS), lambda g, s: (s[1] // 8, 0)),
            pl.BlockSpec((1, _N_ITEMS), lambda g, s: (0, 0)),
            pl.BlockSpec((1, _N_USERS), lambda g, s: (0, 0)),
        ],
        out_specs=pl.BlockSpec((1, 1), lambda g, s: (0, 0)),
        scratch_shapes=[pltpu.VMEM((_N_USERS, _N_ITEMS), jnp.bfloat16),
                        pltpu.VMEM((_N_USERS, 1), jnp.float32)],
    )
    out = pl.pallas_call(
        _nbm_kernel,
        grid_spec=grid_spec,
        out_shape=jax.ShapeDtypeStruct((1, 1), jnp.float32),
    )(sref, R, R, item_weights, implicit_offset, ib, ub)
    return out[0, 0]
